# Initial kernel scaffold; baseline (speedup 1.0000x reference)
import jax
import jax.numpy as jnp
from jax.experimental import pallas as pl


def kernel(positions, pe):
    raise NotImplementedError("write your pallas kernel here")


# speedup vs baseline: 3.0588x; 3.0588x over previous
import functools

import jax
import jax.numpy as jnp
from jax import lax
from jax.experimental import pallas as pl
from jax.experimental.pallas import tpu as pltpu
from jax.experimental.pallas import tpu_sc as plsc

_LANES = 128
_STREAMS = 8
_CHUNK = _LANES * _STREAMS


def _make_gather(N, D, n_workers, per_w):
    n_iters = per_w // _CHUNK
    idx_rows_per_w = per_w // _LANES
    mesh = plsc.VectorSubcoreMesh(core_axis_name="c", subcore_axis_name="s")
    nc = plsc.get_sparse_core_info().num_cores

    @functools.partial(
        pl.kernel,
        mesh=mesh,
        out_type=jax.ShapeDtypeStruct((N, D), jnp.float32),
        scratch_types=[
            pltpu.VMEM((_STREAMS, _LANES), jnp.int32),
            pltpu.VMEM((_CHUNK, D), jnp.float32),
            pltpu.SemaphoreType.DMA,
        ],
        compiler_params=pltpu.CompilerParams(use_tc_tiling_on_sc=False),
    )
    def gather_kernel(table_hbm, idx_hbm, out_hbm, idx_v, rows_v, sem):
        wid = lax.axis_index("s") * nc + lax.axis_index("c")
        idx_row0 = wid * idx_rows_per_w
        out0 = wid * per_w

        def body(g, carry):
            pltpu.sync_copy(
                idx_hbm.at[pl.ds(idx_row0 + g * _STREAMS, _STREAMS)], idx_v)
            copies = []
            for j in range(_STREAMS):
                copies.append(
                    pltpu.async_copy(
                        table_hbm.at[idx_v.at[j]],
                        rows_v.at[pl.ds(j * _LANES, _LANES)],
                        sem,
                    ))
            for c in copies:
                c.wait()
            pltpu.sync_copy(
                rows_v, out_hbm.at[pl.ds(out0 + g * _CHUNK, _CHUNK)])
            return carry

        lax.fori_loop(0, n_iters, body, 0)

    return gather_kernel


def kernel(positions, pe):
    B, H = positions.shape
    V, D = pe.shape
    N = B * H
    n_workers = 32
    per_w = N // n_workers
    idx2d = positions.reshape(N // _LANES, _LANES).astype(jnp.int32)
    out = _make_gather(N, D, n_workers, per_w)(pe, idx2d)
    return out.reshape(B, H, D)


# speedup vs baseline: 3.0722x; 1.0044x over previous
import functools

import jax
import jax.numpy as jnp
from jax import lax
from jax.experimental import pallas as pl
from jax.experimental.pallas import tpu as pltpu
from jax.experimental.pallas import tpu_sc as plsc

_LANES = 128
_STREAMS = 4
_CHUNK = _LANES * _STREAMS


def _make_gather(N, D, n_workers, per_w):
    n_iters = per_w // _CHUNK
    assert n_iters % 2 == 0
    idx_rows_per_w = per_w // _LANES
    mesh = plsc.VectorSubcoreMesh(core_axis_name="c", subcore_axis_name="s")
    nc = plsc.get_sparse_core_info().num_cores
    g_bytes = _STREAMS * _LANES * D * 4
    o_bytes = _CHUNK * D * 4

    @functools.partial(
        pl.kernel,
        mesh=mesh,
        out_type=jax.ShapeDtypeStruct((N, D), jnp.float32),
        scratch_types=[
            pltpu.VMEM((_STREAMS, _LANES), jnp.int32),
            pltpu.VMEM((_STREAMS, _LANES), jnp.int32),
            pltpu.VMEM((_CHUNK, D), jnp.float32),
            pltpu.VMEM((_CHUNK, D), jnp.float32),
            pltpu.SemaphoreType.DMA,
            pltpu.SemaphoreType.DMA,
            pltpu.SemaphoreType.DMA,
            pltpu.SemaphoreType.DMA,
        ],
        compiler_params=pltpu.CompilerParams(use_tc_tiling_on_sc=False),
    )
    def gather_kernel(table_hbm, idx_hbm, out_hbm,
                      idx_v0, idx_v1, rows0, rows1,
                      sem_g0, sem_g1, sem_o0, sem_o1):
        wid = lax.axis_index("s") * nc + lax.axis_index("c")
        idx_row0 = wid * idx_rows_per_w
        out0 = wid * per_w

        def idx_copy(g, buf):
            pltpu.sync_copy(
                idx_hbm.at[pl.ds(idx_row0 + g * _STREAMS, _STREAMS)], buf)

        def fire_gathers(idx_buf, rows_buf, sem):
            for j in range(_STREAMS):
                pltpu.async_copy(
                    table_hbm.at[idx_buf.at[j]],
                    rows_buf.at[pl.ds(j * _LANES, _LANES)],
                    sem,
                )

        def fire_out(g, rows_buf, sem):
            pltpu.async_copy(
                rows_buf, out_hbm.at[pl.ds(out0 + g * _CHUNK, _CHUNK)], sem)

        def wait_gathers(idx_buf, rows_buf, sem):
            for j in range(_STREAMS):
                pltpu.make_async_copy(
                    table_hbm.at[idx_buf.at[j]],
                    rows_buf.at[pl.ds(j * _LANES, _LANES)],
                    sem,
                ).wait()

        def wait_out(rows_buf, sem):
            pltpu.make_async_copy(
                rows_buf, out_hbm.at[pl.ds(out0, _CHUNK)], sem).wait()

        idx_copy(0, idx_v0)
        fire_gathers(idx_v0, rows0, sem_g0)
        idx_copy(1, idx_v1)

        def body(i, carry):
            g0 = 2 * i

            wait_gathers(idx_v0, rows0, sem_g0)

            @pl.when(i > 0)
            def _():
                wait_out(rows1, sem_o1)

            fire_gathers(idx_v1, rows1, sem_g1)
            fire_out(g0, rows0, sem_o0)
            idx_copy(jnp.minimum(g0 + 2, n_iters - 1), idx_v0)

            wait_gathers(idx_v1, rows1, sem_g1)
            wait_out(rows0, sem_o0)
            fire_gathers(idx_v0, rows0, sem_g0)
            fire_out(g0 + 1, rows1, sem_o1)
            idx_copy(jnp.minimum(g0 + 3, n_iters - 1), idx_v1)
            return carry

        lax.fori_loop(0, n_iters // 2, body, 0)
        wait_gathers(idx_v0, rows0, sem_g0)
        wait_out(rows1, sem_o1)

    return gather_kernel


def kernel(positions, pe):
    B, H = positions.shape
    V, D = pe.shape
    N = B * H
    n_workers = 32
    per_w = N // n_workers
    idx2d = positions.reshape(N // _LANES, _LANES).astype(jnp.int32)
    out = _make_gather(N, D, n_workers, per_w)(pe, idx2d)
    return out.reshape(B, H, D)
